# baseline (speedup 1.0000x reference)
import functools
import math

import jax
import jax.numpy as jnp
import numpy as np
from jax import lax
from jax.experimental import pallas as pl
from jax.experimental.pallas import tpu as pltpu
from jax.experimental.pallas import tpu_sc as plsc

_B = 128
_D_IN = 2048
_F = 4096
_QUANTILE = 0.7

_NC, _NS, _LANES = 2, 16, 16
_NW = _NC * _NS
_SLICE = _F // _NW

_INT_MAX = np.int32(2147483647)
_INT_MIN = np.int32(-2147483648)
_HI_INIT = np.int32(0x7F800000)


def _sc_indicator_body(sel_hbm, out_hbm, sel_v, slice_v):
    wid = lax.axis_index("s") * _NC + lax.axis_index("c")
    base = wid * _SLICE
    k_total = sel_hbm.shape[0]
    pltpu.sync_copy(sel_hbm, sel_v)

    zeros16 = jnp.zeros((_LANES,), jnp.int32)

    def _zero(i, carry):
        slice_v[pl.ds(i * _LANES, _LANES)] = zeros16
        return carry

    lax.fori_loop(0, _SLICE // _LANES, _zero, 0)

    ones16 = jnp.ones((_LANES,), jnp.int32)

    def _scatter(j, carry):
        idx = sel_v[pl.ds(j * _LANES, _LANES)]
        loc = idx - base
        m = (loc >= 0) & (loc < _SLICE)
        locc = jnp.clip(loc, 0, _SLICE - 1)
        plsc.store_scatter(slice_v, [locc], ones16, mask=m)
        return carry

    lax.fori_loop(0, k_total // _LANES, _scatter, 0)
    pltpu.sync_copy(slice_v, out_hbm.at[pl.ds(base, _SLICE)])


def _build_indicator(sel_idx):
    kern = pl.kernel(
        _sc_indicator_body,
        out_type=jax.ShapeDtypeStruct((_F,), jnp.int32),
        mesh=plsc.VectorSubcoreMesh(
            core_axis_name="c", subcore_axis_name="s",
            num_cores=_NC, num_subcores=_NS),
        scratch_types=[
            pltpu.VMEM((sel_idx.shape[0],), jnp.int32),
            pltpu.VMEM((_SLICE,), jnp.int32),
        ],
        compiler_params=pltpu.CompilerParams(needs_layout_passes=False),
    )
    return kern(sel_idx)


def _mm1_body(x_ref, w_ref, b_ref, o_ref):
    o_ref[...] = (
        jnp.dot(x_ref[...], w_ref[...], preferred_element_type=jnp.float32)
        + b_ref[...]
    )


def _mm2_body(y_ref, w_ref, b_ref, sel_ref, o_ref):
    z = (
        jnp.dot(y_ref[...], w_ref[...], preferred_element_type=jnp.float32)
        + b_ref[...]
    )
    bits = lax.bitcast_convert_type(z, jnp.int32)
    key = bits ^ ((bits >> 31) & np.int32(0x7FFFFFFF))
    o_ref[...] = jnp.where(sel_ref[...] != 0, key, _INT_MAX)


def _select_body(kk, nblk, keys_ref, y_ref, gt_ref, o_ref, thr_s):
    j = pl.program_id(0)

    @pl.when(j == 0)
    def _bisect():
        keys = keys_ref[...]

        def _one(lo, hi):
            mid = (lo & hi) + ((lo ^ hi) >> 1)
            cnt = jnp.sum((keys <= mid).astype(jnp.int32), axis=1,
                          keepdims=True)
            ge = cnt >= kk
            return jnp.where(ge, lo, mid + 1), jnp.where(ge, mid, hi)

        def _it(_, lohi):
            lo, hi = lohi
            for _u in range(4):
                lo, hi = _one(lo, hi)
            return lo, hi

        lo0 = jnp.full((_B, 1), _INT_MIN, jnp.int32)
        hi0 = jnp.full((_B, 1), _HI_INIT, jnp.int32)
        _, thr = lax.fori_loop(0, 8, _it, (lo0, hi0))
        thr_s[...] = thr

    @pl.when(j > 0)
    def _merge():
        fb = _F // nblk
        col = pl.multiple_of((j - 1) * fb, fb)
        o_ref[...] = jnp.where(keys_ref[:, pl.ds(col, fb)] <= thr_s[...],
                               gt_ref[...], y_ref[...])


def kernel(x, W_orig, b_orig, W_policy, b_policy, ground_truth, sel_idx):
    K = sel_idx.shape[0]
    kk = int(max(1, min(K, 1 + math.floor(_QUANTILE * (K - 1)))))

    is_sel = _build_indicator(sel_idx).reshape(1, _F)
    b_o = b_orig.reshape(1, _F)
    b_p = b_policy.reshape(1, _F)

    nf1 = 4
    f1 = _F // nf1
    y = pl.pallas_call(
        _mm1_body,
        grid=(nf1,),
        in_specs=[
            pl.BlockSpec((_B, _D_IN), lambda j: (0, 0)),
            pl.BlockSpec((_D_IN, f1), lambda j: (0, j)),
            pl.BlockSpec((1, f1), lambda j: (0, j)),
        ],
        out_specs=pl.BlockSpec((_B, f1), lambda j: (0, j)),
        out_shape=jax.ShapeDtypeStruct((_B, _F), jnp.float32),
    )(x, W_orig, b_o)

    nf2 = 8
    f2 = _F // nf2
    keys = pl.pallas_call(
        _mm2_body,
        grid=(nf2,),
        in_specs=[
            pl.BlockSpec((_B, _F), lambda j: (0, 0)),
            pl.BlockSpec((_F, f2), lambda j: (0, j)),
            pl.BlockSpec((1, f2), lambda j: (0, j)),
            pl.BlockSpec((1, f2), lambda j: (0, j)),
        ],
        out_specs=pl.BlockSpec((_B, f2), lambda j: (0, j)),
        out_shape=jax.ShapeDtypeStruct((_B, _F), jnp.int32),
    )(y, W_policy, b_p, is_sel)

    nblk = 8
    fb = _F // nblk
    out = pl.pallas_call(
        functools.partial(_select_body, kk, nblk),
        grid=(nblk + 1,),
        in_specs=[
            pl.BlockSpec((_B, _F), lambda j: (0, 0)),
            pl.BlockSpec((_B, fb), lambda j: (0, jnp.clip(j - 1, 0, nblk - 1))),
            pl.BlockSpec((_B, fb), lambda j: (0, jnp.clip(j - 1, 0, nblk - 1))),
        ],
        out_specs=pl.BlockSpec((_B, fb),
                               lambda j: (0, jnp.clip(j - 1, 0, nblk - 1))),
        out_shape=jax.ShapeDtypeStruct((_B, _F), jnp.float32),
        scratch_shapes=[pltpu.VMEM((_B, 1), jnp.int32)],
    )(keys, y, ground_truth)
    return out


# speedup vs baseline: 1.0443x; 1.0443x over previous
import functools
import math

import jax
import jax.numpy as jnp
import numpy as np
from jax import lax
from jax.experimental import pallas as pl
from jax.experimental.pallas import tpu as pltpu
from jax.experimental.pallas import tpu_sc as plsc

_B = 128
_D_IN = 2048
_F = 4096
_QUANTILE = 0.7

_NC, _NS, _LANES = 2, 16, 16
_NW = _NC * _NS
_SLICE = _F // _NW

_INT_MAX = np.int32(2147483647)
_INT_MIN = np.int32(-2147483648)
_HI_INIT = np.int32(0x7F800000)


def _sc_indicator_body(sel_hbm, out_hbm, sel_v, slice_v):
    wid = lax.axis_index("s") * _NC + lax.axis_index("c")
    base = wid * _SLICE
    k_total = sel_hbm.shape[0]
    pltpu.sync_copy(sel_hbm, sel_v)

    zeros16 = jnp.zeros((_LANES,), jnp.int32)

    def _zero(i, carry):
        slice_v[pl.ds(i * _LANES, _LANES)] = zeros16
        return carry

    lax.fori_loop(0, _SLICE // _LANES, _zero, 0)

    ones16 = jnp.ones((_LANES,), jnp.int32)

    def _scatter(j, carry):
        idx = sel_v[pl.ds(j * _LANES, _LANES)]
        loc = idx - base
        m = (loc >= 0) & (loc < _SLICE)
        locc = jnp.clip(loc, 0, _SLICE - 1)
        plsc.store_scatter(slice_v, [locc], ones16, mask=m)
        return carry

    lax.fori_loop(0, k_total // _LANES, _scatter, 0)
    pltpu.sync_copy(slice_v, out_hbm.at[pl.ds(base, _SLICE)])


def _build_indicator(sel_idx):
    kern = pl.kernel(
        _sc_indicator_body,
        out_type=jax.ShapeDtypeStruct((_F,), jnp.int32),
        mesh=plsc.VectorSubcoreMesh(
            core_axis_name="c", subcore_axis_name="s",
            num_cores=_NC, num_subcores=_NS),
        scratch_types=[
            pltpu.VMEM((sel_idx.shape[0],), jnp.int32),
            pltpu.VMEM((_SLICE,), jnp.int32),
        ],
        compiler_params=pltpu.CompilerParams(needs_layout_passes=False),
    )
    return kern(sel_idx)


def _mm1_body(x_ref, w_ref, b_ref, o_ref):
    o_ref[...] = (
        jnp.dot(x_ref[...], w_ref[...], preferred_element_type=jnp.float32)
        + b_ref[...]
    )


def _mm2_body(y_ref, w_ref, b_ref, sel_ref, o_ref):
    z = (
        jnp.dot(y_ref[...], w_ref[...], preferred_element_type=jnp.float32)
        + b_ref[...]
    )
    bits = lax.bitcast_convert_type(z, jnp.int32)
    key = bits ^ ((bits >> 31) & np.int32(0x7FFFFFFF))
    o_ref[...] = jnp.where(sel_ref[...] != 0, key, _INT_MAX)


def _select_body(kk, keys_ref, y_ref, gt_ref, o_ref):
    keys = keys_ref[...]

    def _it(_, lohi):
        lo, hi = lohi
        mid = (lo & hi) + ((lo ^ hi) >> 1)
        cnt = jnp.sum((keys <= mid).astype(jnp.int32), axis=1, keepdims=True)
        ge = cnt >= kk
        return jnp.where(ge, lo, mid + 1), jnp.where(ge, mid, hi)

    lo0 = jnp.full((_B, 1), _INT_MIN, jnp.int32)
    hi0 = jnp.full((_B, 1), _HI_INIT, jnp.int32)
    _, thr = lax.fori_loop(0, 32, _it, (lo0, hi0))
    o_ref[...] = jnp.where(keys <= thr, gt_ref[...], y_ref[...])


def kernel(x, W_orig, b_orig, W_policy, b_policy, ground_truth, sel_idx):
    K = sel_idx.shape[0]
    kk = int(max(1, min(K, 1 + math.floor(_QUANTILE * (K - 1)))))

    is_sel = _build_indicator(sel_idx).reshape(1, _F)
    b_o = b_orig.reshape(1, _F)
    b_p = b_policy.reshape(1, _F)

    nf1 = 4
    f1 = _F // nf1
    y = pl.pallas_call(
        _mm1_body,
        grid=(nf1,),
        in_specs=[
            pl.BlockSpec((_B, _D_IN), lambda j: (0, 0)),
            pl.BlockSpec((_D_IN, f1), lambda j: (0, j)),
            pl.BlockSpec((1, f1), lambda j: (0, j)),
        ],
        out_specs=pl.BlockSpec((_B, f1), lambda j: (0, j)),
        out_shape=jax.ShapeDtypeStruct((_B, _F), jnp.float32),
    )(x, W_orig, b_o)

    nf2 = 8
    f2 = _F // nf2
    keys = pl.pallas_call(
        _mm2_body,
        grid=(nf2,),
        in_specs=[
            pl.BlockSpec((_B, _F), lambda j: (0, 0)),
            pl.BlockSpec((_F, f2), lambda j: (0, j)),
            pl.BlockSpec((1, f2), lambda j: (0, j)),
            pl.BlockSpec((1, f2), lambda j: (0, j)),
        ],
        out_specs=pl.BlockSpec((_B, f2), lambda j: (0, j)),
        out_shape=jax.ShapeDtypeStruct((_B, _F), jnp.int32),
    )(y, W_policy, b_p, is_sel)

    out = pl.pallas_call(
        functools.partial(_select_body, kk),
        out_shape=jax.ShapeDtypeStruct((_B, _F), jnp.float32),
    )(keys, y, ground_truth)
    return out
